# baseline (speedup 1.0000x reference)
import math
import functools
import jax, jax.numpy as jnp
from jax.experimental import pallas as pl

G = 100
NPG = 100
NP = 128
N = G * NPG
D = 128
NHID = 128
RATIO = 0.5


def _conv1_body(A_ref, X_ref, dis_ref, W_ref, out_ref):
    g = pl.program_id(0)
    A = A_ref[0]
    Xg = X_ref[0]
    dis = dis_ref[0]
    W = W_ref[...]
    row = jax.lax.broadcasted_iota(jnp.int32, (NP, NP), 0)
    col = jax.lax.broadcasted_iota(jnp.int32, (NP, NP), 1)
    eye = jnp.where(row == col, 1.0, 0.0).astype(jnp.float32)
    Ah = A + eye
    disc = jnp.broadcast_to(dis.reshape(NP, 1), (NP, NP))
    disr = jnp.broadcast_to(dis.reshape(1, NP), (NP, NP))
    An = (disc * Ah) * disr
    M = jax.lax.dot(An, Xg, precision=None, preferred_element_type=jnp.float32)
    Y = jax.lax.dot(M, W, precision=None, preferred_element_type=jnp.float32)
    out_ref[0] = jnp.maximum(Y, 0.0)


def _conv1_pallas(A_pad, X_pad, dis_pad, W1):
    return pl.pallas_call(
        _conv1_body,
        grid=(G,),
        in_specs=[
            pl.BlockSpec((1, NP, NP), lambda g: (g, 0, 0)),
            pl.BlockSpec((1, NP, NP), lambda g: (g, 0, 0)),
            pl.BlockSpec((1, 1, NP), lambda g: (g, 0, 0)),
            pl.BlockSpec((NHID, NHID), lambda g: (0, 0)),
        ],
        out_specs=pl.BlockSpec((1, NP, NP), lambda g: (g, 0, 0)),
        out_shape=jax.ShapeDtypeStruct((G, NP, NHID), jnp.float32),
    )(A_pad, X_pad, dis_pad, W1)


def _dense_adj_pad(edge_index):
    ei = edge_index.astype(jnp.int32)
    g = ei[0] // NPG
    s = ei[0] % NPG
    t = ei[1] % NPG
    A = jnp.zeros((G, NP, NP), dtype=jnp.float32).at[g, s, t].add(1.0)
    A = A + jnp.swapaxes(A, 1, 2)
    return A


def _gcn(X, A, W, b):
    n = A.shape[-1]
    Ah = A + jnp.eye(n, dtype=A.dtype)[None]
    d = Ah.sum(-1)
    dis = 1.0 / jnp.sqrt(jnp.clip(d, 1e-6))
    An = dis[:, :, None] * Ah * dis[:, None, :]
    return jnp.einsum('gij,gjd->gid', An, X) @ W + b


def _hgpsl_pool(X, A, ratio):
    deg = jnp.clip(A.sum(-1), 1.0)
    agg = jnp.einsum('gij,gjd->gid', A, X) / deg[..., None]
    score = jnp.abs(X - agg).sum(-1)
    k = int(math.ceil(ratio * X.shape[1]))
    _, idx = jax.lax.top_k(score, k)
    Xp = jnp.take_along_axis(X, idx[..., None], axis=1)
    Ap = jax.vmap(lambda a, i: a[i][:, i])(A, idx)
    return Xp, Ap


def _readout(X):
    return jnp.concatenate([X.max(axis=1), X.mean(axis=1)], axis=-1)


def kernel(x, edge_index, batch, W1, b1, W2, b2, W3, b3):
    A0p = _dense_adj_pad(edge_index)
    X_pad = jnp.pad(x.reshape(G, NPG, D), ((0, 0), (0, NP - NPG), (0, 0)))

    d = A0p[:, :NPG, :].sum(-1) + 1.0
    dis = 1.0 / jnp.sqrt(jnp.clip(d, 1e-6))
    dis_pad = jnp.pad(dis, ((0, 0), (0, NP - NPG)),
                      constant_values=1.0).reshape(G, 1, NP)

    X1p = _conv1_pallas(A0p, X_pad, dis_pad, W1)
    X = X1p[:, :NPG, :]
    A0 = A0p[:, :NPG, :NPG]

    xs0 = X.reshape(-1, NHID)
    b0 = batch

    Xp1, A1 = _hgpsl_pool(X, A0, RATIO)
    x1 = _readout(Xp1)

    X2 = jax.nn.relu(_gcn(Xp1, A1, W2, b2))
    xs2 = X2.reshape(-1, NHID)
    b2_ids = jnp.repeat(jnp.arange(G, dtype=jnp.int32), Xp1.shape[1])

    Xp2, A2 = _hgpsl_pool(X2, A1, RATIO)
    x2 = _readout(Xp2)

    X3 = jax.nn.relu(_gcn(Xp2, A2, W3, b3))
    xs4 = X3.reshape(-1, NHID)
    b4_ids = jnp.repeat(jnp.arange(G, dtype=jnp.int32), Xp2.shape[1])

    x3 = _readout(X3)
    summary = jax.nn.relu(x1) + jax.nn.relu(x2) + jax.nn.relu(x3)
    return (summary, xs0, xs2, xs4, b0, b2_ids, b4_ids)
